# Initial kernel scaffold; baseline (speedup 1.0000x reference)
import jax
import jax.numpy as jnp
from jax.experimental import pallas as pl


def kernel(logits, top_k):
    raise NotImplementedError("write your pallas kernel here")


# speedup vs baseline: 80.3990x; 80.3990x over previous
import jax
import jax.numpy as jnp
from jax.experimental import pallas as pl
from jax.experimental.pallas import tpu as pltpu

_INV_TEMP = 1.25
_TOP_P = 0.9
_NROWS = 128
_VOCAB = 100000
_BLK_R = 8
_INT_MIN = -(2 ** 31)


def _sample_kernel(x_ref, k_ref, probs_ref, tok_ref):
    r, v = x_ref.shape
    l = x_ref[...] * _INV_TEMP
    l = l + 0.0
    i32 = jax.lax.bitcast_convert_type(l, jnp.int32)
    key = jnp.where(i32 < 0, i32 ^ jnp.int32(0x7FFFFFFF), i32)
    kf = k_ref[0, 0]

    def topk_body(it, tau_b):
        bit = 31 - it
        m = jnp.left_shift(jnp.int32(1), bit)
        cand_b = tau_b | m
        cand_s = cand_b ^ _INT_MIN
        cnt = jnp.sum(jnp.where(key >= cand_s, 1.0, 0.0), axis=-1,
                      keepdims=True)
        return jnp.where(cnt >= kf, cand_b, tau_b)

    tau_b = jax.lax.fori_loop(0, 32, topk_body, jnp.zeros((r, 1), jnp.int32))
    tau = tau_b ^ _INT_MIN
    keep = key >= tau

    mval = jnp.max(l, axis=-1, keepdims=True)
    e = jnp.where(keep, jnp.exp(l - mval), 0.0)
    z = jnp.sum(e, axis=-1, keepdims=True)
    pz = z * _TOP_P

    def topp_body(it, t_b):
        bit = 31 - it
        m = jnp.left_shift(jnp.int32(1), bit)
        cand_b = t_b | m
        cand_s = cand_b ^ _INT_MIN
        mass = jnp.sum(jnp.where(key > cand_s, e, 0.0), axis=-1,
                       keepdims=True)
        return jnp.where(mass > pz, cand_b, t_b)

    t_b = jax.lax.fori_loop(0, 32, topp_body, jnp.zeros((r, 1), jnp.int32))
    t2 = t_b ^ _INT_MIN
    keep2 = key > t2

    e2 = jnp.where(keep2, e, 0.0)
    z2 = jnp.sum(e2, axis=-1, keepdims=True)
    probs_ref[...] = e2 / z2

    mkey = jnp.max(key, axis=-1, keepdims=True)
    iota = jax.lax.broadcasted_iota(jnp.int32, (r, v), 1)
    tok = jnp.min(jnp.where(key == mkey, iota, jnp.int32(2 ** 31 - 1)),
                  axis=-1, keepdims=True)
    tok_ref[...] = jnp.broadcast_to(tok, tok_ref.shape)


def kernel(logits, top_k):
    k_arr = jnp.asarray(top_k, jnp.float32).reshape(1, 1)
    nblk = _NROWS // _BLK_R
    probs, tok = pl.pallas_call(
        _sample_kernel,
        grid=(nblk,),
        in_specs=[
            pl.BlockSpec((_BLK_R, _VOCAB), lambda i: (i, 0)),
            pl.BlockSpec((1, 1), lambda i: (0, 0)),
        ],
        out_specs=[
            pl.BlockSpec((_BLK_R, _VOCAB), lambda i: (i, 0)),
            pl.BlockSpec((_BLK_R, 128), lambda i: (i, 0)),
        ],
        out_shape=[
            jax.ShapeDtypeStruct((_NROWS, _VOCAB), jnp.float32),
            jax.ShapeDtypeStruct((_NROWS, 128), jnp.int32),
        ],
    )(logits, k_arr)
    return probs, tok[:, 0]
